# baseline (speedup 1.0000x reference)
import functools

import jax
import jax.numpy as jnp
from jax import lax
from jax.experimental import pallas as pl
from jax.experimental.pallas import tpu as pltpu
from jax.experimental.pallas import tpu_sc as plsc

_DIM = 128
_NC = 2
_NS = 16
_NW = _NC * _NS
_L = 16
_CH = 80


def _rup(a, b):
    return (a + b - 1) // b * b


def _mesh():
    return plsc.VectorSubcoreMesh(core_axis_name="c", subcore_axis_name="s")


def _zero_zbuf(zbuf):
    z = jnp.zeros((_L,), jnp.float32)
    for r in range(16):
        for c in range(_DIM // _L):
            zbuf[r, pl.ds(c * _L, _L)] = z


@functools.lru_cache(maxsize=None)
def _row_agg(n_pad, n_edges):
    per = n_edges // _NW
    nch = per // _CH
    rows_per_tile = n_pad // _NS

    @functools.partial(
        pl.kernel,
        mesh=_mesh(),
        out_type=jax.ShapeDtypeStruct((_NC, n_pad, _DIM), jnp.float32),
        scratch_types=[
            pltpu.VMEM((_CH,), jnp.int32),
            pltpu.VMEM((_CH,), jnp.int32),
            pltpu.VMEM((_CH, _DIM), jnp.float32),
            pltpu.VMEM((16, _DIM), jnp.float32),
            pltpu.VMEM_SHARED((n_pad, _DIM), jnp.float32),
            pltpu.SemaphoreType.DMA,
        ],
    )
    def k(table, src, dst, out, sidx, didx, rows, zbuf, acc, sem):
        cid = lax.axis_index("c")
        sid = lax.axis_index("s")
        wid = sid * _NC + cid
        _zero_zbuf(zbuf)
        rbase = sid * rows_per_tile

        def zloop(i, c):
            pltpu.sync_copy(zbuf, acc.at[pl.ds(rbase + i * 16, 16)])
            return c

        lax.fori_loop(0, rows_per_tile // 16, zloop, 0)
        plsc.subcore_barrier()

        ebase = wid * per

        def body(i, c):
            off = ebase + i * _CH
            pltpu.sync_copy(src.at[pl.ds(off, _CH)], sidx)
            pltpu.sync_copy(dst.at[pl.ds(off, _CH)], didx)
            pltpu.async_copy(table.at[sidx], rows, sem).wait()
            pltpu.sync_copy(rows, acc.at[didx], add=True)
            return c

        lax.fori_loop(0, nch, body, 0)
        plsc.subcore_barrier()

        def cpy(i, c):
            r0 = rbase + i * 16
            pltpu.sync_copy(acc.at[pl.ds(r0, 16)], out.at[cid, pl.ds(r0, 16)])
            return c

        lax.fori_loop(0, rows_per_tile // 16, cpy, 0)

    return k


@functools.lru_cache(maxsize=None)
def _scalar_deg(n_pad, n_edges):
    per = n_edges // _NW
    nit = per // _L

    @functools.partial(
        pl.kernel,
        mesh=_mesh(),
        out_type=jax.ShapeDtypeStruct((_NW, n_pad), jnp.float32),
        compiler_params=pltpu.CompilerParams(needs_layout_passes=False),
        scratch_types=[
            pltpu.VMEM((per,), jnp.int32),
            pltpu.VMEM((n_pad,), jnp.float32),
        ],
    )
    def k(dst, out, dst_v, acc):
        cid = lax.axis_index("c")
        sid = lax.axis_index("s")
        wid = sid * _NC + cid
        pltpu.sync_copy(dst.at[pl.ds(wid * per, per)], dst_v)
        z = jnp.zeros((_L,), jnp.float32)

        def zloop(i, c):
            acc[pl.ds(i * _L, _L)] = z
            return c

        lax.fori_loop(0, n_pad // _L, zloop, 0)
        one = jnp.full((_L,), 1.0, jnp.float32)

        def body(i, c):
            d = dst_v[pl.ds(i * _L, _L)]
            plsc.addupdate_scatter(acc, [d], one)
            return c

        lax.fori_loop(0, nit, body, 0)
        pltpu.sync_copy(acc, out.at[wid])

    return k


@functools.lru_cache(maxsize=None)
def _scalar_num(n_pad, n_edges):
    per = n_edges // _NW
    nit = per // _L

    @functools.partial(
        pl.kernel,
        mesh=_mesh(),
        out_type=jax.ShapeDtypeStruct((_NW, n_pad), jnp.float32),
        compiler_params=pltpu.CompilerParams(needs_layout_passes=False),
        scratch_types=[
            pltpu.VMEM((per,), jnp.int32),
            pltpu.VMEM((per,), jnp.int32),
            pltpu.VMEM((n_pad,), jnp.float32),
            pltpu.VMEM((n_pad,), jnp.float32),
            pltpu.VMEM((n_pad,), jnp.float32),
        ],
    )
    def k(src, dst, h, dinv, out, src_v, dst_v, h_v, di_v, acc):
        cid = lax.axis_index("c")
        sid = lax.axis_index("s")
        wid = sid * _NC + cid
        pltpu.sync_copy(src.at[pl.ds(wid * per, per)], src_v)
        pltpu.sync_copy(dst.at[pl.ds(wid * per, per)], dst_v)
        pltpu.sync_copy(h, h_v)
        pltpu.sync_copy(dinv, di_v)
        z = jnp.zeros((_L,), jnp.float32)

        def zloop(i, c):
            acc[pl.ds(i * _L, _L)] = z
            return c

        lax.fori_loop(0, n_pad // _L, zloop, 0)

        def body(i, c):
            s = src_v[pl.ds(i * _L, _L)]
            d = dst_v[pl.ds(i * _L, _L)]
            hv = plsc.load_gather(h_v, [s])
            a = plsc.load_gather(di_v, [s])
            b = plsc.load_gather(di_v, [d])
            plsc.addupdate_scatter(acc, [d], hv * a * b)
            return c

        lax.fori_loop(0, nit, body, 0)
        pltpu.sync_copy(acc, out.at[wid])

    return k


@functools.lru_cache(maxsize=None)
def _gather_rows(n_pad, k_pad):
    per = k_pad // _NW
    nch = per // _CH

    @functools.partial(
        pl.kernel,
        mesh=_mesh(),
        out_type=jax.ShapeDtypeStruct((k_pad, _DIM), jnp.float32),
        scratch_types=[
            pltpu.VMEM((_CH,), jnp.int32),
            pltpu.VMEM((_CH, _DIM), jnp.float32),
            pltpu.SemaphoreType.DMA,
        ],
    )
    def k(table, idx, out, iv, buf, sem):
        cid = lax.axis_index("c")
        sid = lax.axis_index("s")
        wid = sid * _NC + cid
        base = wid * per

        def body(i, c):
            off = base + i * _CH
            pltpu.sync_copy(idx.at[pl.ds(off, _CH)], iv)
            pltpu.async_copy(table.at[iv], buf, sem).wait()
            pltpu.sync_copy(buf, out.at[pl.ds(off, _CH)])
            return c

        lax.fori_loop(0, nch, body, 0)

    return k


@functools.lru_cache(maxsize=None)
def _scatter_rows(k_pad, n_pad):
    per = k_pad // _NS
    nch = per // _CH
    rows_per_tile = n_pad // _NS

    @functools.partial(
        pl.kernel,
        mesh=_mesh(),
        out_type=jax.ShapeDtypeStruct((n_pad, _DIM), jnp.float32),
        scratch_types=[
            pltpu.VMEM((_CH,), jnp.int32),
            pltpu.VMEM((_CH, _DIM), jnp.float32),
            pltpu.VMEM((16, _DIM), jnp.float32),
            pltpu.VMEM_SHARED((n_pad, _DIM), jnp.float32),
        ],
    )
    def k(rows, idx, out, iv, buf, zbuf, acc, ):
        cid = lax.axis_index("c")
        sid = lax.axis_index("s")
        _zero_zbuf(zbuf)
        rbase = sid * rows_per_tile

        def zloop(i, c):
            pltpu.sync_copy(zbuf, acc.at[pl.ds(rbase + i * 16, 16)])
            return c

        lax.fori_loop(0, rows_per_tile // 16, zloop, 0)
        plsc.subcore_barrier()

        @pl.when(cid == 0)
        def _():
            base = sid * per

            def body(i, c):
                off = base + i * _CH
                pltpu.sync_copy(idx.at[pl.ds(off, _CH)], iv)
                pltpu.sync_copy(rows.at[pl.ds(off, _CH)], buf)
                pltpu.sync_copy(buf, acc.at[iv])
                return c

            lax.fori_loop(0, nch, body, 0)

        plsc.subcore_barrier()

        @pl.when(cid == 0)
        def _():
            def cpy(i, c):
                r0 = rbase + i * 16
                pltpu.sync_copy(acc.at[pl.ds(r0, 16)], out.at[pl.ds(r0, 16)])
                return c

            lax.fori_loop(0, rows_per_tile // 16, cpy, 0)

    return k


def _gin_mlp(x, aparts, w1, b1, w2, b2, g, b):
    n = x.shape[0]

    def body(x_ref, a_ref, w1r, b1r, w2r, b2r, gr, br, o_ref):
        h = x_ref[...] + a_ref[0] + a_ref[1]
        y = jnp.maximum(
            jnp.dot(h, w1r[...], preferred_element_type=jnp.float32) + b1r[...], 0.0)
        z = jnp.maximum(
            jnp.dot(y, w2r[...], preferred_element_type=jnp.float32) + b2r[...], 0.0)
        o_ref[...] = z * gr[...] + br[...]

    return pl.pallas_call(
        body,
        out_shape=jax.ShapeDtypeStruct((n, _DIM), jnp.float32),
    )(x, aparts, w1, b1[None], w2, b2[None], g[None], b[None])


def _score_a(x, w_row, deg_parts):
    n = x.shape[0]

    def body(x_ref, w_ref, dp_ref, h_ref, di_ref):
        h = jnp.sum(x_ref[...] * w_ref[...], axis=1, keepdims=True)
        h_ref[...] = h
        deg = jnp.sum(dp_ref[...], axis=0)[:, None] + 1.0
        di_ref[...] = lax.rsqrt(deg)

    return pl.pallas_call(
        body,
        out_shape=[
            jax.ShapeDtypeStruct((n, 1), jnp.float32),
            jax.ShapeDtypeStruct((n, 1), jnp.float32),
        ],
    )(x, w_row, deg_parts)


def _score_b(num_parts, h, dinv, bias):
    n = h.shape[0]

    def body(np_ref, h_ref, di_ref, b_ref, o_ref):
        s = jnp.sum(np_ref[...], axis=0)[:, None]
        di = di_ref[...]
        o_ref[...] = s + di * di * h_ref[...] + b_ref[0, 0]

    return pl.pallas_call(
        body,
        out_shape=jax.ShapeDtypeStruct((n, 1), jnp.float32),
    )(num_parts, h, dinv, bias.reshape(1, 1))


def _gate_bn(rows, vals, g, b):
    n = rows.shape[0]

    def body(r_ref, v_ref, gr, br, o_ref):
        o_ref[...] = r_ref[...] * jnp.tanh(v_ref[...]) * gr[...] + br[...]

    return pl.pallas_call(
        body,
        out_shape=jax.ShapeDtypeStruct((n, _DIM), jnp.float32),
    )(rows, vals, g[None], b[None])


def _unpool_assemble(s_parts, deg_parts, outfull, nm, xres, rw):
    n = outfull.shape[0]

    def body(sp_ref, dp_ref, of_ref, nm_ref, xr_ref, rw_ref, o_ref):
        s = sp_ref[0] + sp_ref[1]
        deg = jnp.sum(dp_ref[...], axis=0)[:, None]
        mean = s / jnp.maximum(deg, 1.0)
        sel = nm_ref[...] >= 0
        u = jnp.where(sel, of_ref[...], mean)
        o_ref[...] = u + rw_ref[0, 0] * xr_ref[...]

    return pl.pallas_call(
        body,
        out_shape=jax.ShapeDtypeStruct((n, _DIM), jnp.float32),
    )(s_parts, deg_parts, outfull, nm[:, None], xres, rw.reshape(1, 1))


def _readout(x):
    n = x.shape[0]

    def body(x_ref, o_ref):
        xv = x_ref[...]
        o_ref[...] = jnp.concatenate(
            [jnp.max(xv, axis=0), jnp.sum(xv, axis=0) * (1.0 / n)])[None, :]

    return pl.pallas_call(
        body,
        out_shape=jax.ShapeDtypeStruct((1, 2 * _DIM), jnp.float32),
    )(x)


def _head(r2, r3, r4, p):
    def body(r2r, r3r, r4r,
             a1w, a1b, g61, b61,
             a2w, a2b, g62, b62,
             a3w, a3b, g63, b63,
             aw, ab, g6, b6, lw, lb, o_ref):
        def attn_scalars(z, m):
            z = z - jnp.max(z)
            e = jnp.exp(z)
            se = jnp.sum(e)
            i2 = lax.broadcasted_iota(jnp.int32, z.shape, 1)
            return [jnp.sum(jnp.where(i2 == j, e, 0.0)) / se for j in range(m)]

        def gate2(r, awr, abr, gr, br):
            rv = r[...]
            z = jnp.dot(rv, awr[...], preferred_element_type=jnp.float32) + abr[...]
            a0, a1 = attn_scalars(z, 2)
            rr = jnp.concatenate([rv[:, :_DIM] * a0, rv[:, _DIM:] * a1], axis=1)
            return rr * gr[...] + br[...]

        q2 = gate2(r2r, a1w, a1b, g61, b61)
        q3 = gate2(r3r, a2w, a2b, g62, b62)
        q4 = gate2(r4r, a3w, a3b, g63, b63)
        xc = jnp.concatenate([q2, q3, q4], axis=1)
        z = jnp.dot(xc, aw[...], preferred_element_type=jnp.float32) + ab[...]
        a0, a1, a2 = attn_scalars(z, 3)
        td = 2 * _DIM
        xc = jnp.concatenate(
            [xc[:, :td] * a0, xc[:, td:2 * td] * a1, xc[:, 2 * td:] * a2], axis=1)
        xc = xc * g6[...] + b6[...]
        o = jnp.maximum(
            jnp.dot(xc, lw[...], preferred_element_type=jnp.float32) + lb[...], 0.0)
        m = jnp.max(o)
        o = o - m
        o_ref[...] = o - jnp.log(jnp.sum(jnp.exp(o)))

    args = [r2, r3, r4,
            p["attn1_W"], p["attn1_b"][None], p["bn61_g"][None], p["bn61_b"][None],
            p["attn2_W"], p["attn2_b"][None], p["bn62_g"][None], p["bn62_b"][None],
            p["attn3_W"], p["attn3_b"][None], p["bn63_g"][None], p["bn63_b"][None],
            p["attn_W"], p["attn_b"][None], p["bn6_g"][None], p["bn6_b"][None],
            p["lin1_W"], p["lin1_b"][None]]
    return pl.pallas_call(
        body,
        out_shape=jax.ShapeDtypeStruct((1, 10), jnp.float32),
    )(*args)


def _padr(a, n_pad):
    return jnp.pad(a, ((0, n_pad - a.shape[0]), (0, 0)))


def _pad1(a, n_pad, value=0):
    return jnp.pad(a, (0, n_pad - a.shape[0]), constant_values=value)


def kernel(x, edge_index, batch, params):
    p = params
    n0 = x.shape[0]
    ne = edge_index.shape[1]
    k1 = n0 // 2
    k2 = k1 // 2
    ones = jnp.ones((_DIM,), jnp.float32)
    zeros = jnp.zeros((_DIM,), jnp.float32)

    src0 = edge_index[0]
    dst0 = edge_index[1]
    n0p = _rup(n0 + 8, 256)
    n1p = _rup(k1 + 8, 256)
    n2p = _rup(k2 + 8, 256)
    k1p = _rup(k1, _CH * _NW)
    k2p = _rup(k2, _CH * _NW)

    deg0 = _scalar_deg(n0p, ne)(dst0)
    agg0 = _row_agg(n0p, ne)(_padr(x, n0p), src0, dst0)
    x0 = _gin_mlp(x, agg0[:, :n0], p["conv1_1_W"], p["conv1_1_b"],
                  p["conv1_2_W"], p["conv1_2_b"], ones, zeros)

    h0, dinv0 = _score_a(x0, p["pool1_W"][:, 0][None], deg0[:, :n0])
    num0 = _scalar_num(n0p, ne)(src0, dst0, _pad1(h0[:, 0], n0p),
                                _pad1(dinv0[:, 0], n0p))
    score0 = _score_b(num0[:, :n0], h0, dinv0, p["pool1_b"])[:, 0]
    vals1, perm1 = lax.top_k(score0, k1)
    nm1 = jnp.full((n0,), -1, jnp.int32).at[perm1].set(
        jnp.arange(k1, dtype=jnp.int32))
    ns = nm1[src0]
    nd = nm1[dst0]
    v1m = (ns >= 0) & (nd >= 0)
    src1 = jnp.where(v1m, ns, 0)
    dst1 = jnp.where(v1m, nd, k1)

    rows1 = _gather_rows(n0p, k1p)(_padr(x0, n0p), _pad1(perm1, k1p))[:k1]
    x1bn = _gate_bn(rows1, vals1[:, None], p["bn1_g"], p["bn1_b"])

    deg1 = _scalar_deg(n1p, ne)(dst1)
    agg1 = _row_agg(n1p, ne)(_padr(x1bn, n1p), src1, dst1)
    x1c = _gin_mlp(x1bn, agg1[:, :k1], p["conv2_1_W"], p["conv2_1_b"],
                   p["conv2_2_W"], p["conv2_2_b"], ones, zeros)

    h1, dinv1 = _score_a(x1c, p["pool2_W"][:, 0][None], deg1[:, :k1])
    num1 = _scalar_num(n1p, ne)(src1, dst1, _pad1(h1[:, 0], n1p),
                                _pad1(dinv1[:, 0], n1p))
    score1 = _score_b(num1[:, :k1], h1, dinv1, p["pool2_b"])[:, 0]
    vals2, perm2 = lax.top_k(score1, k2)
    nm2 = jnp.full((k1,), -1, jnp.int32).at[perm2].set(
        jnp.arange(k2, dtype=jnp.int32))
    nm2p = _pad1(nm2, n1p, value=-1)
    ns2 = nm2p[src1]
    nd2 = nm2p[dst1]
    v2m = (ns2 >= 0) & (nd2 >= 0)
    src2 = jnp.where(v2m, ns2, 0)
    dst2 = jnp.where(v2m, nd2, k2)

    rows2 = _gather_rows(n1p, k2p)(_padr(x1c, n1p), _pad1(perm2, k2p))[:k2]
    x2bn = _gate_bn(rows2, vals2[:, None], p["bn2_g"], p["bn2_b"])

    agg2 = _row_agg(n2p, ne)(_padr(x2bn, n2p), src2, dst2)
    x2f = _gin_mlp(x2bn, agg2[:, :k2], p["conv3_1_W"], p["conv3_1_b"],
                   p["conv3_2_W"], p["conv3_2_b"], p["bn3_g"], p["bn3_b"])

    out1 = _scatter_rows(k2p, n1p)(_padr(x2f, k2p), _pad1(perm2, k2p, value=k1))
    s1 = _row_agg(n1p, ne)(out1, src1, dst1)
    x3in = _unpool_assemble(s1[:, :k1], deg1[:, :k1], out1[:k1], nm2, x1c,
                            p["rw1"])
    agg3 = _row_agg(n1p, ne)(_padr(x3in, n1p), src1, dst1)
    x3f = _gin_mlp(x3in, agg3[:, :k1], p["conv4_1_W"], p["conv4_1_b"],
                   p["conv4_2_W"], p["conv4_2_b"], p["bn4_g"], p["bn4_b"])

    out2 = _scatter_rows(k1p, n0p)(_padr(x3f, k1p), _pad1(perm1, k1p, value=n0))
    s0 = _row_agg(n0p, ne)(out2, src0, dst0)
    x4in = _unpool_assemble(s0[:, :n0], deg0[:, :n0], out2[:n0], nm1, x0,
                            p["rw2"])
    agg4 = _row_agg(n0p, ne)(_padr(x4in, n0p), src0, dst0)
    x4f = _gin_mlp(x4in, agg4[:, :n0], p["conv5_1_W"], p["conv5_1_b"],
                   p["conv5_2_W"], p["conv5_2_b"], p["bn5_g"], p["bn5_b"])

    r2 = _readout(x2f)
    r3 = _readout(x3f)
    r4 = _readout(x4f)
    return _head(r2, r3, r4, p)


# speedup vs baseline: 1.0026x; 1.0026x over previous
import functools

import jax
import jax.numpy as jnp
from jax import lax
from jax.experimental import pallas as pl
from jax.experimental.pallas import tpu as pltpu
from jax.experimental.pallas import tpu_sc as plsc

_DIM = 128
_NC = 2
_NS = 16
_NW = _NC * _NS
_L = 16
_CH = 80


def _rup(a, b):
    return (a + b - 1) // b * b


def _mesh():
    return plsc.VectorSubcoreMesh(core_axis_name="c", subcore_axis_name="s")


def _zero_zbuf(zbuf):
    z = jnp.zeros((_L,), jnp.float32)
    for r in range(16):
        for c in range(_DIM // _L):
            zbuf[r, pl.ds(c * _L, _L)] = z


_NB = 2


@functools.lru_cache(maxsize=None)
def _row_agg(n_pad, n_edges):
    per = n_edges // _NW
    nch = per // _CH
    ngrp = nch // _NB
    rows_per_tile = n_pad // _NS

    @functools.partial(
        pl.kernel,
        mesh=_mesh(),
        out_type=jax.ShapeDtypeStruct((_NC, n_pad, _DIM), jnp.float32),
        scratch_types=[
            [pltpu.VMEM((_CH,), jnp.int32) for _ in range(_NB)],
            [pltpu.VMEM((_CH,), jnp.int32) for _ in range(_NB)],
            [pltpu.VMEM((_CH, _DIM), jnp.float32) for _ in range(_NB)],
            pltpu.VMEM((16, _DIM), jnp.float32),
            pltpu.VMEM_SHARED((n_pad, _DIM), jnp.float32),
            pltpu.SemaphoreType.DMA,
            pltpu.SemaphoreType.DMA,
            pltpu.SemaphoreType.DMA,
        ],
    )
    def k(table, src, dst, out, sidx, didx, rows, zbuf, acc, isem, gsem, ssem):
        cid = lax.axis_index("c")
        sid = lax.axis_index("s")
        wid = sid * _NC + cid
        _zero_zbuf(zbuf)
        rbase = sid * rows_per_tile

        def zloop(i, c):
            pltpu.sync_copy(zbuf, acc.at[pl.ds(rbase + i * 16, 16)])
            return c

        lax.fori_loop(0, rows_per_tile // 16, zloop, 0)
        plsc.subcore_barrier()

        ebase = wid * per

        def body(g, c):
            off0 = ebase + g * (_NB * _CH)
            for b in range(_NB):
                pltpu.sync_copy(src.at[pl.ds(off0 + b * _CH, _CH)], sidx[b])
                pltpu.sync_copy(dst.at[pl.ds(off0 + b * _CH, _CH)], didx[b])
            gw = [pltpu.async_copy(table.at[sidx[b]], rows[b], gsem)
                  for b in range(_NB)]
            for w in gw:
                w.wait()
            sw = [pltpu.async_copy(rows[b], acc.at[didx[b]], ssem, add=True)
                  for b in range(_NB)]
            for w in sw:
                w.wait()
            return c

        lax.fori_loop(0, ngrp, body, 0)
        for t in range(ngrp * _NB, nch):
            off = ebase + t * _CH
            pltpu.sync_copy(src.at[pl.ds(off, _CH)], sidx[0])
            pltpu.sync_copy(dst.at[pl.ds(off, _CH)], didx[0])
            pltpu.async_copy(table.at[sidx[0]], rows[0], gsem).wait()
            pltpu.sync_copy(rows[0], acc.at[didx[0]], add=True)
        plsc.subcore_barrier()

        def cpy(i, c):
            r0 = rbase + i * 16
            pltpu.sync_copy(acc.at[pl.ds(r0, 16)], out.at[cid, pl.ds(r0, 16)])
            return c

        lax.fori_loop(0, rows_per_tile // 16, cpy, 0)

    return k


@functools.lru_cache(maxsize=None)
def _scalar_deg(n_pad, n_edges):
    per = n_edges // _NW
    nit = per // _L

    @functools.partial(
        pl.kernel,
        mesh=_mesh(),
        out_type=jax.ShapeDtypeStruct((_NW, n_pad), jnp.float32),
        compiler_params=pltpu.CompilerParams(needs_layout_passes=False),
        scratch_types=[
            pltpu.VMEM((per,), jnp.int32),
            pltpu.VMEM((n_pad,), jnp.float32),
        ],
    )
    def k(dst, out, dst_v, acc):
        cid = lax.axis_index("c")
        sid = lax.axis_index("s")
        wid = sid * _NC + cid
        pltpu.sync_copy(dst.at[pl.ds(wid * per, per)], dst_v)
        z = jnp.zeros((_L,), jnp.float32)

        def zloop(i, c):
            acc[pl.ds(i * _L, _L)] = z
            return c

        lax.fori_loop(0, n_pad // _L, zloop, 0)
        one = jnp.full((_L,), 1.0, jnp.float32)

        def body(i, c):
            d = dst_v[pl.ds(i * _L, _L)]
            plsc.addupdate_scatter(acc, [d], one)
            return c

        lax.fori_loop(0, nit, body, 0)
        pltpu.sync_copy(acc, out.at[wid])

    return k


@functools.lru_cache(maxsize=None)
def _scalar_num(n_pad, n_edges):
    per = n_edges // _NW
    nit = per // _L

    @functools.partial(
        pl.kernel,
        mesh=_mesh(),
        out_type=jax.ShapeDtypeStruct((_NW, n_pad), jnp.float32),
        compiler_params=pltpu.CompilerParams(needs_layout_passes=False),
        scratch_types=[
            pltpu.VMEM((per,), jnp.int32),
            pltpu.VMEM((per,), jnp.int32),
            pltpu.VMEM((n_pad,), jnp.float32),
            pltpu.VMEM((n_pad,), jnp.float32),
            pltpu.VMEM((n_pad,), jnp.float32),
        ],
    )
    def k(src, dst, h, dinv, out, src_v, dst_v, h_v, di_v, acc):
        cid = lax.axis_index("c")
        sid = lax.axis_index("s")
        wid = sid * _NC + cid
        pltpu.sync_copy(src.at[pl.ds(wid * per, per)], src_v)
        pltpu.sync_copy(dst.at[pl.ds(wid * per, per)], dst_v)
        pltpu.sync_copy(h, h_v)
        pltpu.sync_copy(dinv, di_v)
        z = jnp.zeros((_L,), jnp.float32)

        def zloop(i, c):
            acc[pl.ds(i * _L, _L)] = z
            return c

        lax.fori_loop(0, n_pad // _L, zloop, 0)

        def body(i, c):
            s = src_v[pl.ds(i * _L, _L)]
            d = dst_v[pl.ds(i * _L, _L)]
            hv = plsc.load_gather(h_v, [s])
            a = plsc.load_gather(di_v, [s])
            b = plsc.load_gather(di_v, [d])
            plsc.addupdate_scatter(acc, [d], hv * a * b)
            return c

        lax.fori_loop(0, nit, body, 0)
        pltpu.sync_copy(acc, out.at[wid])

    return k


@functools.lru_cache(maxsize=None)
def _gather_rows(n_pad, k_pad):
    per = k_pad // _NW
    nch = per // _CH

    @functools.partial(
        pl.kernel,
        mesh=_mesh(),
        out_type=jax.ShapeDtypeStruct((k_pad, _DIM), jnp.float32),
        scratch_types=[
            pltpu.VMEM((_CH,), jnp.int32),
            pltpu.VMEM((_CH, _DIM), jnp.float32),
            pltpu.SemaphoreType.DMA,
        ],
    )
    def k(table, idx, out, iv, buf, sem):
        cid = lax.axis_index("c")
        sid = lax.axis_index("s")
        wid = sid * _NC + cid
        base = wid * per

        def body(i, c):
            off = base + i * _CH
            pltpu.sync_copy(idx.at[pl.ds(off, _CH)], iv)
            pltpu.async_copy(table.at[iv], buf, sem).wait()
            pltpu.sync_copy(buf, out.at[pl.ds(off, _CH)])
            return c

        lax.fori_loop(0, nch, body, 0)

    return k


@functools.lru_cache(maxsize=None)
def _scatter_rows(k_pad, n_pad):
    per = k_pad // _NS
    nch = per // _CH
    rows_per_tile = n_pad // _NS

    @functools.partial(
        pl.kernel,
        mesh=_mesh(),
        out_type=jax.ShapeDtypeStruct((n_pad, _DIM), jnp.float32),
        scratch_types=[
            pltpu.VMEM((_CH,), jnp.int32),
            pltpu.VMEM((_CH, _DIM), jnp.float32),
            pltpu.VMEM((16, _DIM), jnp.float32),
            pltpu.VMEM_SHARED((n_pad, _DIM), jnp.float32),
        ],
    )
    def k(rows, idx, out, iv, buf, zbuf, acc, ):
        cid = lax.axis_index("c")
        sid = lax.axis_index("s")
        _zero_zbuf(zbuf)
        rbase = sid * rows_per_tile

        def zloop(i, c):
            pltpu.sync_copy(zbuf, acc.at[pl.ds(rbase + i * 16, 16)])
            return c

        lax.fori_loop(0, rows_per_tile // 16, zloop, 0)
        plsc.subcore_barrier()

        @pl.when(cid == 0)
        def _():
            base = sid * per

            def body(i, c):
                off = base + i * _CH
                pltpu.sync_copy(idx.at[pl.ds(off, _CH)], iv)
                pltpu.sync_copy(rows.at[pl.ds(off, _CH)], buf)
                pltpu.sync_copy(buf, acc.at[iv])
                return c

            lax.fori_loop(0, nch, body, 0)

        plsc.subcore_barrier()

        @pl.when(cid == 0)
        def _():
            def cpy(i, c):
                r0 = rbase + i * 16
                pltpu.sync_copy(acc.at[pl.ds(r0, 16)], out.at[pl.ds(r0, 16)])
                return c

            lax.fori_loop(0, rows_per_tile // 16, cpy, 0)

    return k


def _gin_mlp(x, aparts, w1, b1, w2, b2, g, b):
    n = x.shape[0]

    def body(x_ref, a_ref, w1r, b1r, w2r, b2r, gr, br, o_ref):
        h = x_ref[...] + a_ref[0] + a_ref[1]
        y = jnp.maximum(
            jnp.dot(h, w1r[...], preferred_element_type=jnp.float32) + b1r[...], 0.0)
        z = jnp.maximum(
            jnp.dot(y, w2r[...], preferred_element_type=jnp.float32) + b2r[...], 0.0)
        o_ref[...] = z * gr[...] + br[...]

    return pl.pallas_call(
        body,
        out_shape=jax.ShapeDtypeStruct((n, _DIM), jnp.float32),
    )(x, aparts, w1, b1[None], w2, b2[None], g[None], b[None])


def _score_a(x, w_row, deg_parts):
    n = x.shape[0]

    def body(x_ref, w_ref, dp_ref, h_ref, di_ref):
        h = jnp.sum(x_ref[...] * w_ref[...], axis=1, keepdims=True)
        h_ref[...] = h
        deg = jnp.sum(dp_ref[...], axis=0)[:, None] + 1.0
        di_ref[...] = lax.rsqrt(deg)

    return pl.pallas_call(
        body,
        out_shape=[
            jax.ShapeDtypeStruct((n, 1), jnp.float32),
            jax.ShapeDtypeStruct((n, 1), jnp.float32),
        ],
    )(x, w_row, deg_parts)


def _score_b(num_parts, h, dinv, bias):
    n = h.shape[0]

    def body(np_ref, h_ref, di_ref, b_ref, o_ref):
        s = jnp.sum(np_ref[...], axis=0)[:, None]
        di = di_ref[...]
        o_ref[...] = s + di * di * h_ref[...] + b_ref[0, 0]

    return pl.pallas_call(
        body,
        out_shape=jax.ShapeDtypeStruct((n, 1), jnp.float32),
    )(num_parts, h, dinv, bias.reshape(1, 1))


def _gate_bn(rows, vals, g, b):
    n = rows.shape[0]

    def body(r_ref, v_ref, gr, br, o_ref):
        o_ref[...] = r_ref[...] * jnp.tanh(v_ref[...]) * gr[...] + br[...]

    return pl.pallas_call(
        body,
        out_shape=jax.ShapeDtypeStruct((n, _DIM), jnp.float32),
    )(rows, vals, g[None], b[None])


def _unpool_assemble(s_parts, deg_parts, outfull, nm, xres, rw):
    n = outfull.shape[0]

    def body(sp_ref, dp_ref, of_ref, nm_ref, xr_ref, rw_ref, o_ref):
        s = sp_ref[0] + sp_ref[1]
        deg = jnp.sum(dp_ref[...], axis=0)[:, None]
        mean = s / jnp.maximum(deg, 1.0)
        sel = nm_ref[...] >= 0
        u = jnp.where(sel, of_ref[...], mean)
        o_ref[...] = u + rw_ref[0, 0] * xr_ref[...]

    return pl.pallas_call(
        body,
        out_shape=jax.ShapeDtypeStruct((n, _DIM), jnp.float32),
    )(s_parts, deg_parts, outfull, nm[:, None], xres, rw.reshape(1, 1))


def _readout(x):
    n = x.shape[0]

    def body(x_ref, o_ref):
        xv = x_ref[...]
        o_ref[...] = jnp.concatenate(
            [jnp.max(xv, axis=0), jnp.sum(xv, axis=0) * (1.0 / n)])[None, :]

    return pl.pallas_call(
        body,
        out_shape=jax.ShapeDtypeStruct((1, 2 * _DIM), jnp.float32),
    )(x)


def _head(r2, r3, r4, p):
    def body(r2r, r3r, r4r,
             a1w, a1b, g61, b61,
             a2w, a2b, g62, b62,
             a3w, a3b, g63, b63,
             aw, ab, g6, b6, lw, lb, o_ref):
        def attn_scalars(z, m):
            z = z - jnp.max(z)
            e = jnp.exp(z)
            se = jnp.sum(e)
            i2 = lax.broadcasted_iota(jnp.int32, z.shape, 1)
            return [jnp.sum(jnp.where(i2 == j, e, 0.0)) / se for j in range(m)]

        def gate2(r, awr, abr, gr, br):
            rv = r[...]
            z = jnp.dot(rv, awr[...], preferred_element_type=jnp.float32) + abr[...]
            a0, a1 = attn_scalars(z, 2)
            rr = jnp.concatenate([rv[:, :_DIM] * a0, rv[:, _DIM:] * a1], axis=1)
            return rr * gr[...] + br[...]

        q2 = gate2(r2r, a1w, a1b, g61, b61)
        q3 = gate2(r3r, a2w, a2b, g62, b62)
        q4 = gate2(r4r, a3w, a3b, g63, b63)
        xc = jnp.concatenate([q2, q3, q4], axis=1)
        z = jnp.dot(xc, aw[...], preferred_element_type=jnp.float32) + ab[...]
        a0, a1, a2 = attn_scalars(z, 3)
        td = 2 * _DIM
        xc = jnp.concatenate(
            [xc[:, :td] * a0, xc[:, td:2 * td] * a1, xc[:, 2 * td:] * a2], axis=1)
        xc = xc * g6[...] + b6[...]
        o = jnp.maximum(
            jnp.dot(xc, lw[...], preferred_element_type=jnp.float32) + lb[...], 0.0)
        m = jnp.max(o)
        o = o - m
        o_ref[...] = o - jnp.log(jnp.sum(jnp.exp(o)))

    args = [r2, r3, r4,
            p["attn1_W"], p["attn1_b"][None], p["bn61_g"][None], p["bn61_b"][None],
            p["attn2_W"], p["attn2_b"][None], p["bn62_g"][None], p["bn62_b"][None],
            p["attn3_W"], p["attn3_b"][None], p["bn63_g"][None], p["bn63_b"][None],
            p["attn_W"], p["attn_b"][None], p["bn6_g"][None], p["bn6_b"][None],
            p["lin1_W"], p["lin1_b"][None]]
    return pl.pallas_call(
        body,
        out_shape=jax.ShapeDtypeStruct((1, 10), jnp.float32),
    )(*args)


def _padr(a, n_pad):
    return jnp.pad(a, ((0, n_pad - a.shape[0]), (0, 0)))


def _pad1(a, n_pad, value=0):
    return jnp.pad(a, (0, n_pad - a.shape[0]), constant_values=value)


def kernel(x, edge_index, batch, params):
    p = params
    n0 = x.shape[0]
    ne = edge_index.shape[1]
    k1 = n0 // 2
    k2 = k1 // 2
    ones = jnp.ones((_DIM,), jnp.float32)
    zeros = jnp.zeros((_DIM,), jnp.float32)

    src0 = edge_index[0]
    dst0 = edge_index[1]
    n0p = _rup(n0 + 8, 256)
    n1p = _rup(k1 + 8, 256)
    n2p = _rup(k2 + 8, 256)
    k1p = _rup(k1, _CH * _NW)
    k2p = _rup(k2, _CH * _NW)

    deg0 = _scalar_deg(n0p, ne)(dst0)
    agg0 = _row_agg(n0p, ne)(_padr(x, n0p), src0, dst0)
    x0 = _gin_mlp(x, agg0[:, :n0], p["conv1_1_W"], p["conv1_1_b"],
                  p["conv1_2_W"], p["conv1_2_b"], ones, zeros)

    h0, dinv0 = _score_a(x0, p["pool1_W"][:, 0][None], deg0[:, :n0])
    num0 = _scalar_num(n0p, ne)(src0, dst0, _pad1(h0[:, 0], n0p),
                                _pad1(dinv0[:, 0], n0p))
    score0 = _score_b(num0[:, :n0], h0, dinv0, p["pool1_b"])[:, 0]
    vals1, perm1 = lax.top_k(score0, k1)
    nm1 = jnp.full((n0,), -1, jnp.int32).at[perm1].set(
        jnp.arange(k1, dtype=jnp.int32))
    ns = nm1[src0]
    nd = nm1[dst0]
    v1m = (ns >= 0) & (nd >= 0)
    src1 = jnp.where(v1m, ns, 0)
    dst1 = jnp.where(v1m, nd, k1)

    rows1 = _gather_rows(n0p, k1p)(_padr(x0, n0p), _pad1(perm1, k1p))[:k1]
    x1bn = _gate_bn(rows1, vals1[:, None], p["bn1_g"], p["bn1_b"])

    deg1 = _scalar_deg(n1p, ne)(dst1)
    agg1 = _row_agg(n1p, ne)(_padr(x1bn, n1p), src1, dst1)
    x1c = _gin_mlp(x1bn, agg1[:, :k1], p["conv2_1_W"], p["conv2_1_b"],
                   p["conv2_2_W"], p["conv2_2_b"], ones, zeros)

    h1, dinv1 = _score_a(x1c, p["pool2_W"][:, 0][None], deg1[:, :k1])
    num1 = _scalar_num(n1p, ne)(src1, dst1, _pad1(h1[:, 0], n1p),
                                _pad1(dinv1[:, 0], n1p))
    score1 = _score_b(num1[:, :k1], h1, dinv1, p["pool2_b"])[:, 0]
    vals2, perm2 = lax.top_k(score1, k2)
    nm2 = jnp.full((k1,), -1, jnp.int32).at[perm2].set(
        jnp.arange(k2, dtype=jnp.int32))
    nm2p = _pad1(nm2, n1p, value=-1)
    ns2 = nm2p[src1]
    nd2 = nm2p[dst1]
    v2m = (ns2 >= 0) & (nd2 >= 0)
    src2 = jnp.where(v2m, ns2, 0)
    dst2 = jnp.where(v2m, nd2, k2)

    rows2 = _gather_rows(n1p, k2p)(_padr(x1c, n1p), _pad1(perm2, k2p))[:k2]
    x2bn = _gate_bn(rows2, vals2[:, None], p["bn2_g"], p["bn2_b"])

    agg2 = _row_agg(n2p, ne)(_padr(x2bn, n2p), src2, dst2)
    x2f = _gin_mlp(x2bn, agg2[:, :k2], p["conv3_1_W"], p["conv3_1_b"],
                   p["conv3_2_W"], p["conv3_2_b"], p["bn3_g"], p["bn3_b"])

    out1 = _scatter_rows(k2p, n1p)(_padr(x2f, k2p), _pad1(perm2, k2p, value=k1))
    s1 = _row_agg(n1p, ne)(out1, src1, dst1)
    x3in = _unpool_assemble(s1[:, :k1], deg1[:, :k1], out1[:k1], nm2, x1c,
                            p["rw1"])
    agg3 = _row_agg(n1p, ne)(_padr(x3in, n1p), src1, dst1)
    x3f = _gin_mlp(x3in, agg3[:, :k1], p["conv4_1_W"], p["conv4_1_b"],
                   p["conv4_2_W"], p["conv4_2_b"], p["bn4_g"], p["bn4_b"])

    out2 = _scatter_rows(k1p, n0p)(_padr(x3f, k1p), _pad1(perm1, k1p, value=n0))
    s0 = _row_agg(n0p, ne)(out2, src0, dst0)
    x4in = _unpool_assemble(s0[:, :n0], deg0[:, :n0], out2[:n0], nm1, x0,
                            p["rw2"])
    agg4 = _row_agg(n0p, ne)(_padr(x4in, n0p), src0, dst0)
    x4f = _gin_mlp(x4in, agg4[:, :n0], p["conv5_1_W"], p["conv5_1_b"],
                   p["conv5_2_W"], p["conv5_2_b"], p["bn5_g"], p["bn5_b"])

    r2 = _readout(x2f)
    r3 = _readout(x3f)
    r4 = _readout(x4f)
    return _head(r2, r3, r4, p)


# speedup vs baseline: 27.5360x; 27.4646x over previous
import functools

import jax
import jax.numpy as jnp
from jax import lax
from jax.experimental import pallas as pl
from jax.experimental.pallas import tpu as pltpu
from jax.experimental.pallas import tpu_sc as plsc

_DIM = 128
_NC = 2
_NS = 16
_NW = _NC * _NS
_L = 16
_CH = 80


def _rup(a, b):
    return (a + b - 1) // b * b


def _mesh():
    return plsc.VectorSubcoreMesh(core_axis_name="c", subcore_axis_name="s")


def _zero_zbuf(zbuf):
    z = jnp.zeros((_L,), jnp.float32)
    for r in range(16):
        for c in range(_DIM // _L):
            zbuf[r, pl.ds(c * _L, _L)] = z


_NB = 2


@functools.lru_cache(maxsize=None)
def _row_agg(n_pad, stride):
    rows_per_tile = n_pad // _NS

    @functools.partial(
        pl.kernel,
        mesh=_mesh(),
        out_type=jax.ShapeDtypeStruct((_NC, n_pad, _DIM), jnp.float32),
        scratch_types=[
            [pltpu.VMEM((_CH,), jnp.int32) for _ in range(_NB)],
            [pltpu.VMEM((_CH,), jnp.int32) for _ in range(_NB)],
            [pltpu.VMEM((_CH, _DIM), jnp.float32) for _ in range(_NB)],
            pltpu.VMEM((16, _DIM), jnp.float32),
            pltpu.VMEM((16,), jnp.int32),
            pltpu.VMEM_SHARED((n_pad, _DIM), jnp.float32),
            pltpu.SemaphoreType.DMA,
            pltpu.SemaphoreType.DMA,
            pltpu.SemaphoreType.DMA,
        ],
    )
    def k(table, src, dst, cnts, out, sidx, didx, rows, zbuf, cv, acc,
          isem, gsem, ssem):
        cid = lax.axis_index("c")
        sid = lax.axis_index("s")
        wid = sid * _NC + cid
        _zero_zbuf(zbuf)
        rbase = sid * rows_per_tile

        def zloop(i, c):
            pltpu.sync_copy(zbuf, acc.at[pl.ds(rbase + i * 16, 16)])
            return c

        lax.fori_loop(0, rows_per_tile // 16, zloop, 0)
        plsc.subcore_barrier()

        pltpu.sync_copy(cnts.at[pl.ds(wid * 16, 16)], cv)
        cnt = cv[...][0]
        nch = cnt // _CH
        ngrp = nch // _NB
        ebase = wid * stride

        def chunk(t, b):
            off = ebase + t * _CH
            pltpu.sync_copy(src.at[pl.ds(off, _CH)], sidx[b])
            pltpu.sync_copy(dst.at[pl.ds(off, _CH)], didx[b])

        def body(g, c):
            t0 = g * _NB
            for b in range(_NB):
                chunk(t0 + b, b)
            gw = [pltpu.async_copy(table.at[sidx[b]], rows[b], gsem)
                  for b in range(_NB)]
            for w in gw:
                w.wait()
            sw = [pltpu.async_copy(rows[b], acc.at[didx[b]], ssem, add=True)
                  for b in range(_NB)]
            for w in sw:
                w.wait()
            return c

        lax.fori_loop(0, ngrp, body, 0)

        def tail(t, c):
            chunk(t, 0)
            pltpu.async_copy(table.at[sidx[0]], rows[0], gsem).wait()
            pltpu.sync_copy(rows[0], acc.at[didx[0]], add=True)
            return c

        lax.fori_loop(ngrp * _NB, nch, tail, 0)
        plsc.subcore_barrier()

        def cpy(i, c):
            r0 = rbase + i * 16
            pltpu.sync_copy(acc.at[pl.ds(r0, 16)], out.at[cid, pl.ds(r0, 16)])
            return c

        lax.fori_loop(0, rows_per_tile // 16, cpy, 0)

    return k


@functools.lru_cache(maxsize=None)
def _scalar_deg(n_pad, stride):
    @functools.partial(
        pl.kernel,
        mesh=_mesh(),
        out_type=jax.ShapeDtypeStruct((_NW, n_pad), jnp.float32),
        compiler_params=pltpu.CompilerParams(needs_layout_passes=False),
        scratch_types=[
            pltpu.VMEM((stride,), jnp.int32),
            pltpu.VMEM((16,), jnp.int32),
            pltpu.VMEM((n_pad,), jnp.float32),
        ],
    )
    def k(dst, cnts, out, dst_v, cv, acc):
        cid = lax.axis_index("c")
        sid = lax.axis_index("s")
        wid = sid * _NC + cid
        pltpu.sync_copy(dst.at[pl.ds(wid * stride, stride)], dst_v)
        pltpu.sync_copy(cnts.at[pl.ds(wid * 16, 16)], cv)
        cnt = cv[...][0]
        z = jnp.zeros((_L,), jnp.float32)

        def zloop(i, c):
            acc[pl.ds(i * _L, _L)] = z
            return c

        lax.fori_loop(0, n_pad // _L, zloop, 0)
        one = jnp.full((_L,), 1.0, jnp.float32)

        def body(i, c):
            d = dst_v[pl.ds(i * _L, _L)]
            plsc.addupdate_scatter(acc, [d], one)
            return c

        lax.fori_loop(0, cnt // _L, body, 0)
        pltpu.sync_copy(acc, out.at[wid])

    return k


@functools.lru_cache(maxsize=None)
def _scalar_num(n_pad, stride):
    @functools.partial(
        pl.kernel,
        mesh=_mesh(),
        out_type=jax.ShapeDtypeStruct((_NW, n_pad), jnp.float32),
        compiler_params=pltpu.CompilerParams(needs_layout_passes=False),
        scratch_types=[
            pltpu.VMEM((stride,), jnp.int32),
            pltpu.VMEM((stride,), jnp.int32),
            pltpu.VMEM((16,), jnp.int32),
            pltpu.VMEM((n_pad,), jnp.float32),
            pltpu.VMEM((n_pad,), jnp.float32),
            pltpu.VMEM((n_pad,), jnp.float32),
        ],
    )
    def k(src, dst, cnts, h, dinv, out, src_v, dst_v, cv, h_v, di_v, acc):
        cid = lax.axis_index("c")
        sid = lax.axis_index("s")
        wid = sid * _NC + cid
        pltpu.sync_copy(src.at[pl.ds(wid * stride, stride)], src_v)
        pltpu.sync_copy(dst.at[pl.ds(wid * stride, stride)], dst_v)
        pltpu.sync_copy(cnts.at[pl.ds(wid * 16, 16)], cv)
        cnt = cv[...][0]
        pltpu.sync_copy(h, h_v)
        pltpu.sync_copy(dinv, di_v)
        z = jnp.zeros((_L,), jnp.float32)

        def zloop(i, c):
            acc[pl.ds(i * _L, _L)] = z
            return c

        lax.fori_loop(0, n_pad // _L, zloop, 0)

        def body(i, c):
            s = src_v[pl.ds(i * _L, _L)]
            d = dst_v[pl.ds(i * _L, _L)]
            hv = plsc.load_gather(h_v, [s])
            a = plsc.load_gather(di_v, [s])
            b = plsc.load_gather(di_v, [d])
            plsc.addupdate_scatter(acc, [d], hv * a * b)
            return c

        lax.fori_loop(0, cnt // _L, body, 0)
        pltpu.sync_copy(acc, out.at[wid])

    return k


@functools.lru_cache(maxsize=None)
def _remap_compact(n_prev_pad, stride_in, trash_out):
    stride_out = stride_in + _CH

    @functools.partial(
        pl.kernel,
        mesh=_mesh(),
        out_type=(
            jax.ShapeDtypeStruct((_NW * stride_out,), jnp.int32),
            jax.ShapeDtypeStruct((_NW * stride_out,), jnp.int32),
            jax.ShapeDtypeStruct((_NW * 16,), jnp.int32),
        ),
        compiler_params=pltpu.CompilerParams(needs_layout_passes=False),
        scratch_types=[
            pltpu.VMEM((stride_in,), jnp.int32),
            pltpu.VMEM((stride_in,), jnp.int32),
            pltpu.VMEM((stride_out,), jnp.int32),
            pltpu.VMEM((stride_out,), jnp.int32),
            pltpu.VMEM((n_prev_pad,), jnp.int32),
            pltpu.VMEM((16,), jnp.int32),
        ],
    )
    def k(src, dst, cnts, nm, so, do, co, src_v, dst_v, so_v, do_v, nm_v, cv):
        cid = lax.axis_index("c")
        sid = lax.axis_index("s")
        wid = sid * _NC + cid
        pltpu.sync_copy(src.at[pl.ds(wid * stride_in, stride_in)], src_v)
        pltpu.sync_copy(dst.at[pl.ds(wid * stride_in, stride_in)], dst_v)
        pltpu.sync_copy(nm, nm_v)
        pltpu.sync_copy(cnts.at[pl.ds(wid * 16, 16)], cv)
        cnt_in = cv[...][0]

        def body(i, off):
            s = src_v[pl.ds(i * _L, _L)]
            d = dst_v[pl.ds(i * _L, _L)]
            ns = plsc.load_gather(nm_v, [s])
            nd = plsc.load_gather(nm_v, [d])
            m = (ns >= 0) & (nd >= 0)
            pos = plsc.cumsum(m.astype(jnp.int32)) + (off - 1)
            plsc.store_scatter(so_v, [pos], jnp.where(m, ns, 0), mask=m)
            plsc.store_scatter(do_v, [pos], jnp.where(m, nd, 0), mask=m)
            return jnp.max(pos) + 1

        off_fin = lax.fori_loop(0, cnt_in // _L, body, 0)

        iota = lax.iota(jnp.int32, _L)
        zv = jnp.zeros((_L,), jnp.int32)
        tv = jnp.full((_L,), trash_out, jnp.int32)
        for j in range(_CH // _L):
            pos = off_fin + j * _L + iota
            plsc.store_scatter(so_v, [pos], zv)
            plsc.store_scatter(do_v, [pos], tv)
        cnt_out = (off_fin + _CH - 1) // _CH * _CH
        cv[...] = jnp.full((_L,), 1, jnp.int32) * cnt_out
        pltpu.sync_copy(so_v, so.at[pl.ds(wid * stride_out, stride_out)])
        pltpu.sync_copy(do_v, do.at[pl.ds(wid * stride_out, stride_out)])
        pltpu.sync_copy(cv, co.at[pl.ds(wid * 16, 16)])

    return k


@functools.lru_cache(maxsize=None)
def _gather_rows(n_pad, k_pad):
    per = k_pad // _NW
    nch = per // _CH

    @functools.partial(
        pl.kernel,
        mesh=_mesh(),
        out_type=jax.ShapeDtypeStruct((k_pad, _DIM), jnp.float32),
        scratch_types=[
            pltpu.VMEM((_CH,), jnp.int32),
            pltpu.VMEM((_CH, _DIM), jnp.float32),
            pltpu.SemaphoreType.DMA,
        ],
    )
    def k(table, idx, out, iv, buf, sem):
        cid = lax.axis_index("c")
        sid = lax.axis_index("s")
        wid = sid * _NC + cid
        base = wid * per

        def body(i, c):
            off = base + i * _CH
            pltpu.sync_copy(idx.at[pl.ds(off, _CH)], iv)
            pltpu.async_copy(table.at[iv], buf, sem).wait()
            pltpu.sync_copy(buf, out.at[pl.ds(off, _CH)])
            return c

        lax.fori_loop(0, nch, body, 0)

    return k


@functools.lru_cache(maxsize=None)
def _scatter_rows(k_pad, n_pad):
    per = k_pad // _NS
    nch = per // _CH
    rows_per_tile = n_pad // _NS

    @functools.partial(
        pl.kernel,
        mesh=_mesh(),
        out_type=jax.ShapeDtypeStruct((n_pad, _DIM), jnp.float32),
        scratch_types=[
            pltpu.VMEM((_CH,), jnp.int32),
            pltpu.VMEM((_CH, _DIM), jnp.float32),
            pltpu.VMEM((16, _DIM), jnp.float32),
            pltpu.VMEM_SHARED((n_pad, _DIM), jnp.float32),
        ],
    )
    def k(rows, idx, out, iv, buf, zbuf, acc, ):
        cid = lax.axis_index("c")
        sid = lax.axis_index("s")
        _zero_zbuf(zbuf)
        rbase = sid * rows_per_tile

        def zloop(i, c):
            pltpu.sync_copy(zbuf, acc.at[pl.ds(rbase + i * 16, 16)])
            return c

        lax.fori_loop(0, rows_per_tile // 16, zloop, 0)
        plsc.subcore_barrier()

        @pl.when(cid == 0)
        def _():
            base = sid * per

            def body(i, c):
                off = base + i * _CH
                pltpu.sync_copy(idx.at[pl.ds(off, _CH)], iv)
                pltpu.sync_copy(rows.at[pl.ds(off, _CH)], buf)
                pltpu.sync_copy(buf, acc.at[iv])
                return c

            lax.fori_loop(0, nch, body, 0)

        plsc.subcore_barrier()

        @pl.when(cid == 0)
        def _():
            def cpy(i, c):
                r0 = rbase + i * 16
                pltpu.sync_copy(acc.at[pl.ds(r0, 16)], out.at[pl.ds(r0, 16)])
                return c

            lax.fori_loop(0, rows_per_tile // 16, cpy, 0)

    return k


def _gin_mlp(x, aparts, w1, b1, w2, b2, g, b):
    n = x.shape[0]

    def body(x_ref, a_ref, w1r, b1r, w2r, b2r, gr, br, o_ref):
        h = x_ref[...] + a_ref[0] + a_ref[1]
        y = jnp.maximum(
            jnp.dot(h, w1r[...], preferred_element_type=jnp.float32) + b1r[...], 0.0)
        z = jnp.maximum(
            jnp.dot(y, w2r[...], preferred_element_type=jnp.float32) + b2r[...], 0.0)
        o_ref[...] = z * gr[...] + br[...]

    return pl.pallas_call(
        body,
        out_shape=jax.ShapeDtypeStruct((n, _DIM), jnp.float32),
    )(x, aparts, w1, b1[None], w2, b2[None], g[None], b[None])


def _score_a(x, w_row, deg_parts):
    n = x.shape[0]

    def body(x_ref, w_ref, dp_ref, h_ref, di_ref):
        h = jnp.sum(x_ref[...] * w_ref[...], axis=1, keepdims=True)
        h_ref[...] = h
        deg = jnp.sum(dp_ref[...], axis=0)[:, None] + 1.0
        di_ref[...] = lax.rsqrt(deg)

    return pl.pallas_call(
        body,
        out_shape=[
            jax.ShapeDtypeStruct((n, 1), jnp.float32),
            jax.ShapeDtypeStruct((n, 1), jnp.float32),
        ],
    )(x, w_row, deg_parts)


def _score_b(num_parts, h, dinv, bias):
    n = h.shape[0]

    def body(np_ref, h_ref, di_ref, b_ref, o_ref):
        s = jnp.sum(np_ref[...], axis=0)[:, None]
        di = di_ref[...]
        o_ref[...] = s + di * di * h_ref[...] + b_ref[0, 0]

    return pl.pallas_call(
        body,
        out_shape=jax.ShapeDtypeStruct((n, 1), jnp.float32),
    )(num_parts, h, dinv, bias.reshape(1, 1))


def _gate_bn(rows, vals, g, b):
    n = rows.shape[0]

    def body(r_ref, v_ref, gr, br, o_ref):
        o_ref[...] = r_ref[...] * jnp.tanh(v_ref[...]) * gr[...] + br[...]

    return pl.pallas_call(
        body,
        out_shape=jax.ShapeDtypeStruct((n, _DIM), jnp.float32),
    )(rows, vals, g[None], b[None])


def _unpool_assemble(s_parts, deg_parts, outfull, nm, xres, rw):
    n = outfull.shape[0]

    def body(sp_ref, dp_ref, of_ref, nm_ref, xr_ref, rw_ref, o_ref):
        s = sp_ref[0] + sp_ref[1]
        deg = jnp.sum(dp_ref[...], axis=0)[:, None]
        mean = s / jnp.maximum(deg, 1.0)
        sel = nm_ref[...] >= 0
        u = jnp.where(sel, of_ref[...], mean)
        o_ref[...] = u + rw_ref[0, 0] * xr_ref[...]

    return pl.pallas_call(
        body,
        out_shape=jax.ShapeDtypeStruct((n, _DIM), jnp.float32),
    )(s_parts, deg_parts, outfull, nm[:, None], xres, rw.reshape(1, 1))


def _readout(x):
    n = x.shape[0]

    def body(x_ref, o_ref):
        xv = x_ref[...]
        o_ref[...] = jnp.concatenate(
            [jnp.max(xv, axis=0), jnp.sum(xv, axis=0) * (1.0 / n)])[None, :]

    return pl.pallas_call(
        body,
        out_shape=jax.ShapeDtypeStruct((1, 2 * _DIM), jnp.float32),
    )(x)


def _head(r2, r3, r4, p):
    def body(r2r, r3r, r4r,
             a1w, a1b, g61, b61,
             a2w, a2b, g62, b62,
             a3w, a3b, g63, b63,
             aw, ab, g6, b6, lw, lb, o_ref):
        def attn_scalars(z, m):
            z = z - jnp.max(z)
            e = jnp.exp(z)
            se = jnp.sum(e)
            i2 = lax.broadcasted_iota(jnp.int32, z.shape, 1)
            return [jnp.sum(jnp.where(i2 == j, e, 0.0)) / se for j in range(m)]

        def gate2(r, awr, abr, gr, br):
            rv = r[...]
            z = jnp.dot(rv, awr[...], preferred_element_type=jnp.float32) + abr[...]
            a0, a1 = attn_scalars(z, 2)
            rr = jnp.concatenate([rv[:, :_DIM] * a0, rv[:, _DIM:] * a1], axis=1)
            return rr * gr[...] + br[...]

        q2 = gate2(r2r, a1w, a1b, g61, b61)
        q3 = gate2(r3r, a2w, a2b, g62, b62)
        q4 = gate2(r4r, a3w, a3b, g63, b63)
        xc = jnp.concatenate([q2, q3, q4], axis=1)
        z = jnp.dot(xc, aw[...], preferred_element_type=jnp.float32) + ab[...]
        a0, a1, a2 = attn_scalars(z, 3)
        td = 2 * _DIM
        xc = jnp.concatenate(
            [xc[:, :td] * a0, xc[:, td:2 * td] * a1, xc[:, 2 * td:] * a2], axis=1)
        xc = xc * g6[...] + b6[...]
        o = jnp.maximum(
            jnp.dot(xc, lw[...], preferred_element_type=jnp.float32) + lb[...], 0.0)
        m = jnp.max(o)
        o = o - m
        o_ref[...] = o - jnp.log(jnp.sum(jnp.exp(o)))

    args = [r2, r3, r4,
            p["attn1_W"], p["attn1_b"][None], p["bn61_g"][None], p["bn61_b"][None],
            p["attn2_W"], p["attn2_b"][None], p["bn62_g"][None], p["bn62_b"][None],
            p["attn3_W"], p["attn3_b"][None], p["bn63_g"][None], p["bn63_b"][None],
            p["attn_W"], p["attn_b"][None], p["bn6_g"][None], p["bn6_b"][None],
            p["lin1_W"], p["lin1_b"][None]]
    return pl.pallas_call(
        body,
        out_shape=jax.ShapeDtypeStruct((1, 10), jnp.float32),
    )(*args)


def _padr(a, n_pad):
    return jnp.pad(a, ((0, n_pad - a.shape[0]), (0, 0)))


def _pad1(a, n_pad, value=0):
    return jnp.pad(a, (0, n_pad - a.shape[0]), constant_values=value)


def kernel(x, edge_index, batch, params):
    p = params
    n0 = x.shape[0]
    ne = edge_index.shape[1]
    k1 = n0 // 2
    k2 = k1 // 2
    ones = jnp.ones((_DIM,), jnp.float32)
    zeros = jnp.zeros((_DIM,), jnp.float32)

    src0 = edge_index[0]
    dst0 = edge_index[1]
    n0p = _rup(n0 + 8, 256)
    n1p = _rup(k1 + 8, 256)
    n2p = _rup(k2 + 8, 256)
    k1p = _rup(k1, _CH * _NW)
    k2p = _rup(k2, _CH * _NW)
    per0 = ne // _NW
    st1 = per0 + _CH
    st2 = st1 + _CH
    cnts0 = jnp.full((_NW * 16,), per0, jnp.int32)

    deg0 = _scalar_deg(n0p, per0)(dst0, cnts0)
    agg0 = _row_agg(n0p, per0)(_padr(x, n0p), src0, dst0, cnts0)
    x0 = _gin_mlp(x, agg0[:, :n0], p["conv1_1_W"], p["conv1_1_b"],
                  p["conv1_2_W"], p["conv1_2_b"], ones, zeros)

    h0, dinv0 = _score_a(x0, p["pool1_W"][:, 0][None], deg0[:, :n0])
    num0 = _scalar_num(n0p, per0)(src0, dst0, cnts0, _pad1(h0[:, 0], n0p),
                                  _pad1(dinv0[:, 0], n0p))
    score0 = _score_b(num0[:, :n0], h0, dinv0, p["pool1_b"])[:, 0]
    vals1, perm1 = lax.top_k(score0, k1)
    nm1 = jnp.full((n0,), -1, jnp.int32).at[perm1].set(
        jnp.arange(k1, dtype=jnp.int32))
    src1b, dst1b, cnts1 = _remap_compact(n0p, per0, k1)(
        src0, dst0, cnts0, _pad1(nm1, n0p, value=-1))

    rows1 = _gather_rows(n0p, k1p)(_padr(x0, n0p), _pad1(perm1, k1p))[:k1]
    x1bn = _gate_bn(rows1, vals1[:, None], p["bn1_g"], p["bn1_b"])

    deg1 = _scalar_deg(n1p, st1)(dst1b, cnts1)
    agg1 = _row_agg(n1p, st1)(_padr(x1bn, n1p), src1b, dst1b, cnts1)
    x1c = _gin_mlp(x1bn, agg1[:, :k1], p["conv2_1_W"], p["conv2_1_b"],
                   p["conv2_2_W"], p["conv2_2_b"], ones, zeros)

    h1, dinv1 = _score_a(x1c, p["pool2_W"][:, 0][None], deg1[:, :k1])
    num1 = _scalar_num(n1p, st1)(src1b, dst1b, cnts1, _pad1(h1[:, 0], n1p),
                                 _pad1(dinv1[:, 0], n1p))
    score1 = _score_b(num1[:, :k1], h1, dinv1, p["pool2_b"])[:, 0]
    vals2, perm2 = lax.top_k(score1, k2)
    nm2 = jnp.full((k1,), -1, jnp.int32).at[perm2].set(
        jnp.arange(k2, dtype=jnp.int32))
    src2b, dst2b, cnts2 = _remap_compact(n1p, st1, k2)(
        src1b, dst1b, cnts1, _pad1(nm2, n1p, value=-1))

    rows2 = _gather_rows(n1p, k2p)(_padr(x1c, n1p), _pad1(perm2, k2p))[:k2]
    x2bn = _gate_bn(rows2, vals2[:, None], p["bn2_g"], p["bn2_b"])

    agg2 = _row_agg(n2p, st2)(_padr(x2bn, n2p), src2b, dst2b, cnts2)
    x2f = _gin_mlp(x2bn, agg2[:, :k2], p["conv3_1_W"], p["conv3_1_b"],
                   p["conv3_2_W"], p["conv3_2_b"], p["bn3_g"], p["bn3_b"])

    out1 = _scatter_rows(k2p, n1p)(_padr(x2f, k2p), _pad1(perm2, k2p, value=k1))
    s1 = _row_agg(n1p, st1)(out1, src1b, dst1b, cnts1)
    x3in = _unpool_assemble(s1[:, :k1], deg1[:, :k1], out1[:k1], nm2, x1c,
                            p["rw1"])
    agg3 = _row_agg(n1p, st1)(_padr(x3in, n1p), src1b, dst1b, cnts1)
    x3f = _gin_mlp(x3in, agg3[:, :k1], p["conv4_1_W"], p["conv4_1_b"],
                   p["conv4_2_W"], p["conv4_2_b"], p["bn4_g"], p["bn4_b"])

    out2 = _scatter_rows(k1p, n0p)(_padr(x3f, k1p), _pad1(perm1, k1p, value=n0))
    s0 = _row_agg(n0p, per0)(out2, src0, dst0, cnts0)
    x4in = _unpool_assemble(s0[:, :n0], deg0[:, :n0], out2[:n0], nm1, x0,
                            p["rw2"])
    agg4 = _row_agg(n0p, per0)(_padr(x4in, n0p), src0, dst0, cnts0)
    x4f = _gin_mlp(x4in, agg4[:, :n0], p["conv5_1_W"], p["conv5_1_b"],
                   p["conv5_2_W"], p["conv5_2_b"], p["bn5_g"], p["bn5_b"])

    r2 = _readout(x2f)
    r3 = _readout(x3f)
    r4 = _readout(x4f)
    return _head(r2, r3, r4, p)
